# baseline (speedup 1.0000x reference)
import jax
import jax.numpy as jnp
from jax.experimental import pallas as pl
from jax.experimental.pallas import tpu as pltpu


def _sum_body(x_ref, o_ref):
    o_ref[...] = jnp.sum(x_ref[...], axis=-1, keepdims=True,
                         dtype=jnp.float32)


def kernel(x_nchw, conv_weight):
    B, C, H, W = x_nchw.shape
    HW = H * W
    del conv_weight
    x = x_nchw.reshape(B, C, HW)

    sums = pl.pallas_call(
        _sum_body,
        out_shape=jax.ShapeDtypeStruct((B, C, 1), jnp.float32),
        grid=(B, 8),
        in_specs=[pl.BlockSpec((None, C // 8, HW), lambda b, t: (b, t, 0))],
        out_specs=pl.BlockSpec((None, C // 8, 1), lambda b, t: (b, t, 0)),
        compiler_params=pltpu.CompilerParams(
            dimension_semantics=("parallel", "arbitrary"),
            vmem_limit_bytes=40 * 1024 * 1024,
        ),
    )(x)

    return sums
